# Initial kernel scaffold; baseline (speedup 1.0000x reference)
import jax
import jax.numpy as jnp
from jax.experimental import pallas as pl


def kernel(x, vel, loc, mes, adj, vel_W1, vel_b1, vel_W2, vel_b2, pos_W1, pos_b1, pos_W2, pos_b2):
    raise NotImplementedError("write your pallas kernel here")


# speedup vs baseline: 6.4258x; 6.4258x over previous
import functools

import jax
import jax.numpy as jnp
from jax import lax
from jax.experimental import pallas as pl
from jax.experimental.pallas import tpu as pltpu
from jax.experimental.pallas import tpu_sc as plsc

_L = 16
_NC = 2
_NS = 16
_NW = _NC * _NS


def _mlp_body(inp_ref, w1_ref, b1_ref, w2_ref, b2_ref, out_ref):
    h = jnp.dot(inp_ref[...], w1_ref[...], preferred_element_type=jnp.float32)
    h = h + b1_ref[...]
    h = h * jax.nn.sigmoid(h)
    out_ref[...] = (
        jnp.dot(h, w2_ref[...], preferred_element_type=jnp.float32) + b2_ref[...]
    )


def _row_mlp(inp, W1, b1, W2, b2, block_r):
    R, D = inp.shape
    return pl.pallas_call(
        _mlp_body,
        grid=(R // block_r,),
        in_specs=[
            pl.BlockSpec((block_r, D), lambda i: (i, 0)),
            pl.BlockSpec((D, D), lambda i: (0, 0)),
            pl.BlockSpec((1, D), lambda i: (0, 0)),
            pl.BlockSpec((D, 1), lambda i: (0, 0)),
            pl.BlockSpec((1, 1), lambda i: (0, 0)),
        ],
        out_specs=pl.BlockSpec((block_r, 1), lambda i: (i, 0)),
        out_shape=jax.ShapeDtypeStruct((R, 1), jnp.float32),
    )(inp, W1, b1.reshape(1, D), W2, b2.reshape(1, 1))


def _combine_body(loc_ref, vel_ref, vm_ref, px_ref, py_ref, pz_ref, out_ref):
    vm = vm_ref[0]
    for i, pr in enumerate((px_ref, py_ref, pz_ref)):
        out_ref[i, :] = loc_ref[i] + pr[0] + pr[1] + vm * vel_ref[i]


def _combine(locT, velT, vmT, px, py, pz):
    npad = locT.shape[1]
    return pl.pallas_call(
        _combine_body,
        out_shape=jax.ShapeDtypeStruct((3, npad), jnp.float32),
    )(locT, velT, vmT, px, py, pz)


@functools.lru_cache(maxsize=None)
def _make_sc_scatter(E, npad):
    pw = E // _NW
    nchunk = pw // _L
    rows = npad // _NS

    mesh = plsc.VectorSubcoreMesh(core_axis_name="c", subcore_axis_name="s")

    def body(locx_hbm, locy_hbm, locz_hbm, send_hbm, rec_hbm, pos_hbm,
             zero_hbm, outx_hbm, outy_hbm, outz_hbm,
             locx_v, locy_v, locz_v, send_v, rec_v, pos_v,
             wx, wy, wz, accx, accy, accz):
        c = lax.axis_index("c")
        s = lax.axis_index("s")
        wid = s * _NC + c
        base = wid * pw

        pltpu.sync_copy(locx_hbm, locx_v)
        pltpu.sync_copy(locy_hbm, locy_v)
        pltpu.sync_copy(locz_hbm, locz_v)
        pltpu.sync_copy(send_hbm.at[pl.ds(base, pw)], send_v)
        pltpu.sync_copy(rec_hbm.at[pl.ds(base, pw)], rec_v)
        pltpu.sync_copy(pos_hbm.at[pl.ds(base, pw)], pos_v)

        @pl.when(s == 0)
        def _init():
            pltpu.sync_copy(zero_hbm, accx)
            pltpu.sync_copy(zero_hbm, accy)
            pltpu.sync_copy(zero_hbm, accz)

        plsc.subcore_barrier()

        def step(i, carry):
            o = pl.multiple_of(i * _L, _L)
            sv = send_v[pl.ds(o, _L)]
            rv = rec_v[pl.ds(o, _L)]
            p = pos_v[pl.ds(o, _L)]
            wx[...] = (plsc.load_gather(locx_v, [sv])
                       - plsc.load_gather(locx_v, [rv])) * p
            wy[...] = (plsc.load_gather(locy_v, [sv])
                       - plsc.load_gather(locy_v, [rv])) * p
            wz[...] = (plsc.load_gather(locz_v, [sv])
                       - plsc.load_gather(locz_v, [rv])) * p
            pltpu.sync_copy(wx, accx.at[rv], add=True)
            pltpu.sync_copy(wy, accy.at[rv], add=True)
            pltpu.sync_copy(wz, accz.at[rv], add=True)
            return carry

        lax.fori_loop(0, nchunk, step, 0)

        plsc.subcore_barrier()

        @pl.when(s == 0)
        def _writeout():
            o = c * npad
            pltpu.sync_copy(accx, outx_hbm.at[pl.ds(o, npad)])
            pltpu.sync_copy(accy, outy_hbm.at[pl.ds(o, npad)])
            pltpu.sync_copy(accz, outz_hbm.at[pl.ds(o, npad)])

    return pl.kernel(
        body,
        out_type=[jax.ShapeDtypeStruct((_NC * npad,), jnp.float32)] * 3,
        mesh=mesh,
        compiler_params=pltpu.CompilerParams(needs_layout_passes=False),
        scratch_types=[
            pltpu.VMEM((npad,), jnp.float32),
            pltpu.VMEM((npad,), jnp.float32),
            pltpu.VMEM((npad,), jnp.float32),
            pltpu.VMEM((pw,), jnp.int32),
            pltpu.VMEM((pw,), jnp.int32),
            pltpu.VMEM((pw,), jnp.float32),
            pltpu.VMEM((_L,), jnp.float32),
            pltpu.VMEM((_L,), jnp.float32),
            pltpu.VMEM((_L,), jnp.float32),
            pltpu.VMEM_SHARED((npad,), jnp.float32),
            pltpu.VMEM_SHARED((npad,), jnp.float32),
            pltpu.VMEM_SHARED((npad,), jnp.float32),
        ],
    )


def kernel(x, vel, loc, mes, adj,
           vel_W1, vel_b1, vel_W2, vel_b2,
           pos_W1, pos_b1, pos_W2, pos_b2):
    N, D = x.shape
    E = mes.shape[0]
    npad = ((N + 127) // 128) * 128

    adj32 = adj.astype(jnp.int32)
    send = adj32[0]
    rec = adj32[1]

    pos_msg = _row_mlp(mes, pos_W1, pos_b1, pos_W2, pos_b2, block_r=2560)
    pos_flat = pos_msg.reshape(E)
    vm = _row_mlp(x, vel_W1, vel_b1, vel_W2, vel_b2, block_r=2000)

    locT = jnp.pad(loc.T, ((0, 0), (0, npad - N)))
    velT = jnp.pad(vel.T, ((0, 0), (0, npad - N)))
    vmT = jnp.pad(vm.T, ((0, 0), (0, npad - N)))
    zero = jnp.zeros((npad,), jnp.float32)

    outx, outy, outz = _make_sc_scatter(E, npad)(
        locT[0], locT[1], locT[2], send, rec, pos_flat, zero)

    outT = _combine(locT, velT, vmT,
                    outx.reshape(_NC, npad),
                    outy.reshape(_NC, npad),
                    outz.reshape(_NC, npad))
    return outT[:, :N].T


# speedup vs baseline: 8.0464x; 1.2522x over previous
import functools

import jax
import jax.numpy as jnp
from jax import lax
from jax.experimental import pallas as pl
from jax.experimental.pallas import tpu as pltpu
from jax.experimental.pallas import tpu_sc as plsc

_L = 16
_B = 128
_NC = 2
_NS = 16
_NW = _NC * _NS


def _mlp_body(inp_ref, w1_ref, b1_ref, w2_ref, b2_ref, out_ref):
    h = jnp.dot(inp_ref[...], w1_ref[...], preferred_element_type=jnp.float32)
    h = h + b1_ref[...]
    h = h * jax.nn.sigmoid(h)
    out_ref[...] = (
        jnp.dot(h, w2_ref[...], preferred_element_type=jnp.float32) + b2_ref[...]
    )


def _row_mlp(inp, W1, b1, W2, b2, block_r):
    R, D = inp.shape
    return pl.pallas_call(
        _mlp_body,
        grid=(R // block_r,),
        in_specs=[
            pl.BlockSpec((block_r, D), lambda i: (i, 0)),
            pl.BlockSpec((D, D), lambda i: (0, 0)),
            pl.BlockSpec((1, D), lambda i: (0, 0)),
            pl.BlockSpec((D, 1), lambda i: (0, 0)),
            pl.BlockSpec((1, 1), lambda i: (0, 0)),
        ],
        out_specs=pl.BlockSpec((block_r, 1), lambda i: (i, 0)),
        out_shape=jax.ShapeDtypeStruct((R, 1), jnp.float32),
    )(inp, W1, b1.reshape(1, D), W2, b2.reshape(1, 1))


def _combine_body(loc_ref, vel_ref, vm_ref, px_ref, py_ref, pz_ref, out_ref):
    vm = vm_ref[0]
    for i, pr in enumerate((px_ref, py_ref, pz_ref)):
        out_ref[i, :] = loc_ref[i] + pr[0] + pr[1] + vm * vel_ref[i]


def _combine(locT, velT, vmT, px, py, pz):
    npad = locT.shape[1]
    return pl.pallas_call(
        _combine_body,
        out_shape=jax.ShapeDtypeStruct((3, npad), jnp.float32),
    )(locT, velT, vmT, px, py, pz)


@functools.lru_cache(maxsize=None)
def _make_sc_scatter(epad, npad):
    pw = epad // _NW
    nchunk = pw // _B
    nsub = _B // _L

    mesh = plsc.VectorSubcoreMesh(core_axis_name="c", subcore_axis_name="s")

    def body(locx_hbm, locy_hbm, locz_hbm, send_hbm, rec2_hbm, pos_hbm,
             zero_hbm, outx_hbm, outy_hbm, outz_hbm,
             locx_v, locy_v, locz_v, send_v, rec2_v, pos_v,
             wx, wy, wz, accx, accy, accz):
        c = lax.axis_index("c")
        s = lax.axis_index("s")
        wid = s * _NC + c
        base = wid * pw

        pltpu.sync_copy(locx_hbm, locx_v)
        pltpu.sync_copy(locy_hbm, locy_v)
        pltpu.sync_copy(locz_hbm, locz_v)
        pltpu.sync_copy(send_hbm.at[pl.ds(base, pw)], send_v)
        pltpu.sync_copy(rec2_hbm.at[pl.ds(wid * nchunk, nchunk)], rec2_v)
        pltpu.sync_copy(pos_hbm.at[pl.ds(base, pw)], pos_v)

        @pl.when(s == 0)
        def _init():
            pltpu.sync_copy(zero_hbm, accx)
            pltpu.sync_copy(zero_hbm, accy)
            pltpu.sync_copy(zero_hbm, accz)

        plsc.subcore_barrier()

        def step(g, carry):
            o = pl.multiple_of(g * _B, _B)
            for j in range(nsub):
                sv = send_v[pl.ds(o + j * _L, _L)]
                rv = rec2_v[g, pl.ds(j * _L, _L)]
                p = pos_v[pl.ds(o + j * _L, _L)]
                wx[pl.ds(j * _L, _L)] = (plsc.load_gather(locx_v, [sv])
                                         - plsc.load_gather(locx_v, [rv])) * p
                wy[pl.ds(j * _L, _L)] = (plsc.load_gather(locy_v, [sv])
                                         - plsc.load_gather(locy_v, [rv])) * p
                wz[pl.ds(j * _L, _L)] = (plsc.load_gather(locz_v, [sv])
                                         - plsc.load_gather(locz_v, [rv])) * p
            idx = rec2_v.at[g]
            pltpu.sync_copy(wx, accx.at[idx], add=True)
            pltpu.sync_copy(wy, accy.at[idx], add=True)
            pltpu.sync_copy(wz, accz.at[idx], add=True)
            return carry

        lax.fori_loop(0, nchunk, step, 0)

        plsc.subcore_barrier()

        @pl.when(s == 0)
        def _writeout():
            o = c * npad
            pltpu.sync_copy(accx, outx_hbm.at[pl.ds(o, npad)])
            pltpu.sync_copy(accy, outy_hbm.at[pl.ds(o, npad)])
            pltpu.sync_copy(accz, outz_hbm.at[pl.ds(o, npad)])

    return pl.kernel(
        body,
        out_type=[jax.ShapeDtypeStruct((_NC * npad,), jnp.float32)] * 3,
        mesh=mesh,
        compiler_params=pltpu.CompilerParams(needs_layout_passes=False),
        scratch_types=[
            pltpu.VMEM((npad,), jnp.float32),
            pltpu.VMEM((npad,), jnp.float32),
            pltpu.VMEM((npad,), jnp.float32),
            pltpu.VMEM((pw,), jnp.int32),
            pltpu.VMEM((nchunk, _B), jnp.int32),
            pltpu.VMEM((pw,), jnp.float32),
            pltpu.VMEM((_B,), jnp.float32),
            pltpu.VMEM((_B,), jnp.float32),
            pltpu.VMEM((_B,), jnp.float32),
            pltpu.VMEM_SHARED((npad,), jnp.float32),
            pltpu.VMEM_SHARED((npad,), jnp.float32),
            pltpu.VMEM_SHARED((npad,), jnp.float32),
        ],
    )


def kernel(x, vel, loc, mes, adj,
           vel_W1, vel_b1, vel_W2, vel_b2,
           pos_W1, pos_b1, pos_W2, pos_b2):
    N, D = x.shape
    E = mes.shape[0]
    npad = ((N + 127) // 128) * 128
    chunk = 8 * _B * _NW
    epad = ((E + chunk - 1) // chunk) * chunk

    adj32 = adj.astype(jnp.int32)
    pad = epad - E
    send = jnp.pad(adj32[0], (0, pad))
    rec = jnp.concatenate(
        [adj32[1], jnp.arange(pad, dtype=jnp.int32) % jnp.int32(N)])

    pos_msg = _row_mlp(mes, pos_W1, pos_b1, pos_W2, pos_b2, block_r=2560)
    pos_flat = jnp.pad(pos_msg.reshape(E), (0, pad))
    vm = _row_mlp(x, vel_W1, vel_b1, vel_W2, vel_b2, block_r=2000)

    locT = jnp.pad(loc.T, ((0, 0), (0, npad - N)))
    velT = jnp.pad(vel.T, ((0, 0), (0, npad - N)))
    vmT = jnp.pad(vm.T, ((0, 0), (0, npad - N)))
    zero = jnp.zeros((npad,), jnp.float32)

    outx, outy, outz = _make_sc_scatter(epad, npad)(
        locT[0], locT[1], locT[2], send, rec.reshape(epad // _B, _B),
        pos_flat, zero)

    outT = _combine(locT, velT, vmT,
                    outx.reshape(_NC, npad),
                    outy.reshape(_NC, npad),
                    outz.reshape(_NC, npad))
    return outT[:, :N].T


# speedup vs baseline: 10.0016x; 1.2430x over previous
import functools

import jax
import jax.numpy as jnp
from jax import lax
from jax.experimental import pallas as pl
from jax.experimental.pallas import tpu as pltpu
from jax.experimental.pallas import tpu_sc as plsc

_L = 16
_B = 128
_NC = 2
_NS = 16
_NW = _NC * _NS


def _mlp_body(inp_ref, w1_ref, b1_ref, w2_ref, b2_ref, out_ref):
    h = jnp.dot(inp_ref[...], w1_ref[...], preferred_element_type=jnp.float32)
    h = h + b1_ref[...]
    h = h * jax.nn.sigmoid(h)
    out_ref[...] = (
        jnp.dot(h, w2_ref[...], preferred_element_type=jnp.float32) + b2_ref[...]
    )


def _row_mlp(inp, W1, b1, W2, b2, block_r):
    R, D = inp.shape
    return pl.pallas_call(
        _mlp_body,
        grid=(R // block_r,),
        in_specs=[
            pl.BlockSpec((block_r, D), lambda i: (i, 0)),
            pl.BlockSpec((D, D), lambda i: (0, 0)),
            pl.BlockSpec((1, D), lambda i: (0, 0)),
            pl.BlockSpec((D, 1), lambda i: (0, 0)),
            pl.BlockSpec((1, 1), lambda i: (0, 0)),
        ],
        out_specs=pl.BlockSpec((block_r, 1), lambda i: (i, 0)),
        out_shape=jax.ShapeDtypeStruct((R, 1), jnp.float32),
    )(inp, W1, b1.reshape(1, D), W2, b2.reshape(1, 1))


def _combine_body(loc_ref, vel_ref, vm_ref, px_ref, py_ref, pz_ref, out_ref):
    vm = vm_ref[0]
    for i, pr in enumerate((px_ref, py_ref, pz_ref)):
        out_ref[i, :] = loc_ref[i] + pr[0] + pr[1] + vm * vel_ref[i]


def _combine(locT, velT, vmT, px, py, pz):
    npad = locT.shape[1]
    return pl.pallas_call(
        _combine_body,
        out_shape=jax.ShapeDtypeStruct((3, npad), jnp.float32),
    )(locT, velT, vmT, px, py, pz)


@functools.lru_cache(maxsize=None)
def _make_sc_scatter(epad, npad):
    pw = epad // _NW
    nchunk = pw // _B
    nsub = _B // _L

    mesh = plsc.VectorSubcoreMesh(core_axis_name="c", subcore_axis_name="s")

    def body(locx_hbm, locy_hbm, locz_hbm, send_hbm, rec2_hbm, pos_hbm,
             zero_hbm, outx_hbm, outy_hbm, outz_hbm,
             locx_v, locy_v, locz_v, send_v, rec2_v, pos_v,
             wx, wy, wz, accx, accy, accz):
        c = lax.axis_index("c")
        s = lax.axis_index("s")
        wid = s * _NC + c
        base = wid * pw

        pltpu.sync_copy(locx_hbm, locx_v)
        pltpu.sync_copy(locy_hbm, locy_v)
        pltpu.sync_copy(locz_hbm, locz_v)
        pltpu.sync_copy(send_hbm.at[pl.ds(base, pw)], send_v)
        pltpu.sync_copy(rec2_hbm.at[pl.ds(wid * nchunk, nchunk)], rec2_v)
        pltpu.sync_copy(pos_hbm.at[pl.ds(base, pw)], pos_v)

        @pl.when(s == 0)
        def _init():
            pltpu.sync_copy(zero_hbm, accx)
            pltpu.sync_copy(zero_hbm, accy)
            pltpu.sync_copy(zero_hbm, accz)

        plsc.subcore_barrier()

        def step(g, carry):
            o = pl.multiple_of(g * _B, _B)
            for j in range(nsub):
                sv = send_v[pl.ds(o + j * _L, _L)]
                rv = rec2_v[g, pl.ds(j * _L, _L)]
                p = pos_v[pl.ds(o + j * _L, _L)]
                wx[pl.ds(j * _L, _L)] = (plsc.load_gather(locx_v, [sv])
                                         - plsc.load_gather(locx_v, [rv])) * p
                wy[pl.ds(j * _L, _L)] = (plsc.load_gather(locy_v, [sv])
                                         - plsc.load_gather(locy_v, [rv])) * p
                wz[pl.ds(j * _L, _L)] = (plsc.load_gather(locz_v, [sv])
                                         - plsc.load_gather(locz_v, [rv])) * p
            idx = rec2_v.at[g]
            pltpu.sync_copy(wx, accx.at[idx], add=True)
            pltpu.sync_copy(wy, accy.at[idx], add=True)
            pltpu.sync_copy(wz, accz.at[idx], add=True)
            return carry

        lax.fori_loop(0, nchunk, step, 0)

        plsc.subcore_barrier()

        @pl.when(s == 0)
        def _writeout():
            o = c * npad
            pltpu.sync_copy(accx, outx_hbm.at[pl.ds(o, npad)])
            pltpu.sync_copy(accy, outy_hbm.at[pl.ds(o, npad)])
            pltpu.sync_copy(accz, outz_hbm.at[pl.ds(o, npad)])

    return pl.kernel(
        body,
        out_type=[jax.ShapeDtypeStruct((_NC * npad,), jnp.float32)] * 3,
        mesh=mesh,
        compiler_params=pltpu.CompilerParams(needs_layout_passes=False),
        scratch_types=[
            pltpu.VMEM((npad,), jnp.float32),
            pltpu.VMEM((npad,), jnp.float32),
            pltpu.VMEM((npad,), jnp.float32),
            pltpu.VMEM((pw,), jnp.int32),
            pltpu.VMEM((nchunk, _B), jnp.int32),
            pltpu.VMEM((pw,), jnp.float32),
            pltpu.VMEM((_B,), jnp.float32),
            pltpu.VMEM((_B,), jnp.float32),
            pltpu.VMEM((_B,), jnp.float32),
            pltpu.VMEM_SHARED((npad,), jnp.float32),
            pltpu.VMEM_SHARED((npad,), jnp.float32),
            pltpu.VMEM_SHARED((npad,), jnp.float32),
        ],
    )


def kernel(x, vel, loc, mes, adj,
           vel_W1, vel_b1, vel_W2, vel_b2,
           pos_W1, pos_b1, pos_W2, pos_b2):
    N, D = x.shape
    E = mes.shape[0]
    npad = ((N + 127) // 128) * 128
    chunk = 8 * _B * _NW
    epad = ((E + chunk - 1) // chunk) * chunk

    adj32 = adj.astype(jnp.int32)
    pad = epad - E
    send = jnp.pad(adj32[0], (0, pad))
    rec = jnp.concatenate(
        [adj32[1], jnp.arange(pad, dtype=jnp.int32) % jnp.int32(N)])

    pos_msg = _row_mlp(mes, pos_W1, pos_b1, pos_W2, pos_b2, block_r=12800)
    pos_flat = jnp.pad(pos_msg.reshape(E), (0, pad))
    vm = _row_mlp(x, vel_W1, vel_b1, vel_W2, vel_b2, block_r=2000)

    locT = jnp.pad(loc.T, ((0, 0), (0, npad - N)))
    velT = jnp.pad(vel.T, ((0, 0), (0, npad - N)))
    vmT = jnp.pad(vm.T, ((0, 0), (0, npad - N)))
    zero = jnp.zeros((npad,), jnp.float32)

    outx, outy, outz = _make_sc_scatter(epad, npad)(
        locT[0], locT[1], locT[2], send, rec.reshape(epad // _B, _B),
        pos_flat, zero)

    outT = _combine(locT, velT, vmT,
                    outx.reshape(_NC, npad),
                    outy.reshape(_NC, npad),
                    outz.reshape(_NC, npad))
    return outT[:, :N].T


# speedup vs baseline: 10.9701x; 1.0968x over previous
import functools

import jax
import jax.numpy as jnp
from jax import lax
from jax.experimental import pallas as pl
from jax.experimental.pallas import tpu as pltpu
from jax.experimental.pallas import tpu_sc as plsc

_L = 16
_B = 128
_NC = 2
_NS = 16
_NW = _NC * _NS


def _mlp_body(inp_ref, w1_ref, b1_ref, w2_ref, b2_ref, out_ref):
    h = jnp.dot(inp_ref[...], w1_ref[...], preferred_element_type=jnp.float32)
    h = h + b1_ref[...]
    h = h * jax.nn.sigmoid(h)
    out_ref[...] = (
        jnp.dot(h, w2_ref[...], preferred_element_type=jnp.float32) + b2_ref[...]
    )


def _row_mlp(inp, W1, b1, W2, b2, block_r):
    R, D = inp.shape
    return pl.pallas_call(
        _mlp_body,
        grid=(R // block_r,),
        in_specs=[
            pl.BlockSpec((block_r, D), lambda i: (i, 0)),
            pl.BlockSpec((D, D), lambda i: (0, 0)),
            pl.BlockSpec((1, D), lambda i: (0, 0)),
            pl.BlockSpec((D, 1), lambda i: (0, 0)),
            pl.BlockSpec((1, 1), lambda i: (0, 0)),
        ],
        out_specs=pl.BlockSpec((block_r, 1), lambda i: (i, 0)),
        out_shape=jax.ShapeDtypeStruct((R, 1), jnp.float32),
    )(inp, W1, b1.reshape(1, D), W2, b2.reshape(1, 1))


def _combine_body(loc_ref, vel_ref, vm_ref, px_ref, py_ref, pz_ref, out_ref):
    vm = vm_ref[0]
    for i, pr in enumerate((px_ref, py_ref, pz_ref)):
        out_ref[i, :] = loc_ref[i] + pr[0] + pr[1] + vm * vel_ref[i]


def _combine(locT, velT, vmT, px, py, pz):
    npad = locT.shape[1]
    return pl.pallas_call(
        _combine_body,
        out_shape=jax.ShapeDtypeStruct((3, npad), jnp.float32),
    )(locT, velT, vmT, px, py, pz)


@functools.lru_cache(maxsize=None)
def _make_sc_scatter(epad, npad):
    pw = epad // _NW
    nchunk = pw // _B
    nsub = _B // _L

    mesh = plsc.VectorSubcoreMesh(core_axis_name="c", subcore_axis_name="s")

    assert nchunk >= 2 and nchunk % 2 == 0

    def body(locx_hbm, locy_hbm, locz_hbm, send_hbm, rec2_hbm, pos_hbm,
             zero_hbm, outx_hbm, outy_hbm, outz_hbm,
             locx_v, locy_v, locz_v, send_v, rec2_v, pos_v,
             wxa, wya, wza, wxb, wyb, wzb, accx, accy, accz,
             sema, semb):
        c = lax.axis_index("c")
        s = lax.axis_index("s")
        wid = s * _NC + c
        base = wid * pw

        pltpu.sync_copy(locx_hbm, locx_v)
        pltpu.sync_copy(locy_hbm, locy_v)
        pltpu.sync_copy(locz_hbm, locz_v)
        pltpu.sync_copy(send_hbm.at[pl.ds(base, pw)], send_v)
        pltpu.sync_copy(rec2_hbm.at[pl.ds(wid * nchunk, nchunk)], rec2_v)
        pltpu.sync_copy(pos_hbm.at[pl.ds(base, pw)], pos_v)

        @pl.when(s == 0)
        def _init():
            pltpu.sync_copy(zero_hbm, accx)
            pltpu.sync_copy(zero_hbm, accy)
            pltpu.sync_copy(zero_hbm, accz)

        plsc.subcore_barrier()

        dummy = zero_hbm.at[pl.ds(0, _B)]

        def compute(g, wx, wy, wz):
            o = pl.multiple_of(g * _B, _B)
            for j in range(nsub):
                sv = send_v[pl.ds(o + j * _L, _L)]
                rv = rec2_v[g, pl.ds(j * _L, _L)]
                p = pos_v[pl.ds(o + j * _L, _L)]
                wx[pl.ds(j * _L, _L)] = (plsc.load_gather(locx_v, [sv])
                                         - plsc.load_gather(locx_v, [rv])) * p
                wy[pl.ds(j * _L, _L)] = (plsc.load_gather(locy_v, [sv])
                                         - plsc.load_gather(locy_v, [rv])) * p
                wz[pl.ds(j * _L, _L)] = (plsc.load_gather(locz_v, [sv])
                                         - plsc.load_gather(locz_v, [rv])) * p

        def fire(g, wx, wy, wz, sem):
            idx = rec2_v.at[g]
            pltpu.async_copy(wx, accx.at[idx], sem, add=True)
            pltpu.async_copy(wy, accy.at[idx], sem, add=True)
            pltpu.async_copy(wz, accz.at[idx], sem, add=True)

        def drain(wx, wy, wz, sem):
            pltpu.make_async_copy(dummy, wx, sem).wait()
            pltpu.make_async_copy(dummy, wy, sem).wait()
            pltpu.make_async_copy(dummy, wz, sem).wait()

        compute(0, wxa, wya, wza)
        fire(0, wxa, wya, wza, sema)

        def step(gg, carry):
            g1 = gg * 2 + 1
            g2 = gg * 2 + 2
            compute(g1, wxb, wyb, wzb)
            fire(g1, wxb, wyb, wzb, semb)
            drain(wxa, wya, wza, sema)
            compute(g2, wxa, wya, wza)
            fire(g2, wxa, wya, wza, sema)
            drain(wxb, wyb, wzb, semb)
            return carry

        lax.fori_loop(0, (nchunk - 2) // 2, step, 0)

        compute(nchunk - 1, wxb, wyb, wzb)
        fire(nchunk - 1, wxb, wyb, wzb, semb)
        drain(wxa, wya, wza, sema)
        drain(wxb, wyb, wzb, semb)

        plsc.subcore_barrier()

        @pl.when(s == 0)
        def _writeout():
            o = c * npad
            pltpu.sync_copy(accx, outx_hbm.at[pl.ds(o, npad)])
            pltpu.sync_copy(accy, outy_hbm.at[pl.ds(o, npad)])
            pltpu.sync_copy(accz, outz_hbm.at[pl.ds(o, npad)])

    return pl.kernel(
        body,
        out_type=[jax.ShapeDtypeStruct((_NC * npad,), jnp.float32)] * 3,
        mesh=mesh,
        compiler_params=pltpu.CompilerParams(needs_layout_passes=False),
        scratch_types=[
            pltpu.VMEM((npad,), jnp.float32),
            pltpu.VMEM((npad,), jnp.float32),
            pltpu.VMEM((npad,), jnp.float32),
            pltpu.VMEM((pw,), jnp.int32),
            pltpu.VMEM((nchunk, _B), jnp.int32),
            pltpu.VMEM((pw,), jnp.float32),
            pltpu.VMEM((_B,), jnp.float32),
            pltpu.VMEM((_B,), jnp.float32),
            pltpu.VMEM((_B,), jnp.float32),
            pltpu.VMEM((_B,), jnp.float32),
            pltpu.VMEM((_B,), jnp.float32),
            pltpu.VMEM((_B,), jnp.float32),
            pltpu.VMEM_SHARED((npad,), jnp.float32),
            pltpu.VMEM_SHARED((npad,), jnp.float32),
            pltpu.VMEM_SHARED((npad,), jnp.float32),
            pltpu.SemaphoreType.DMA,
            pltpu.SemaphoreType.DMA,
        ],
    )


def kernel(x, vel, loc, mes, adj,
           vel_W1, vel_b1, vel_W2, vel_b2,
           pos_W1, pos_b1, pos_W2, pos_b2):
    N, D = x.shape
    E = mes.shape[0]
    npad = ((N + 127) // 128) * 128
    chunk = 8 * _B * _NW
    epad = ((E + chunk - 1) // chunk) * chunk

    adj32 = adj.astype(jnp.int32)
    pad = epad - E
    send = jnp.pad(adj32[0], (0, pad))
    rec = jnp.concatenate(
        [adj32[1], jnp.arange(pad, dtype=jnp.int32) % jnp.int32(N)])

    pos_msg = _row_mlp(mes, pos_W1, pos_b1, pos_W2, pos_b2, block_r=12800)
    pos_flat = jnp.pad(pos_msg.reshape(E), (0, pad))
    vm = _row_mlp(x, vel_W1, vel_b1, vel_W2, vel_b2, block_r=2000)

    locT = jnp.pad(loc.T, ((0, 0), (0, npad - N)))
    velT = jnp.pad(vel.T, ((0, 0), (0, npad - N)))
    vmT = jnp.pad(vm.T, ((0, 0), (0, npad - N)))
    zero = jnp.zeros((npad,), jnp.float32)

    outx, outy, outz = _make_sc_scatter(epad, npad)(
        locT[0], locT[1], locT[2], send, rec.reshape(epad // _B, _B),
        pos_flat, zero)

    outT = _combine(locT, velT, vmT,
                    outx.reshape(_NC, npad),
                    outy.reshape(_NC, npad),
                    outz.reshape(_NC, npad))
    return outT[:, :N].T
